# Initial kernel scaffold; baseline (speedup 1.0000x reference)
import jax
import jax.numpy as jnp
from jax.experimental import pallas as pl


def kernel(input_len, table):
    raise NotImplementedError("write your pallas kernel here")


# speedup vs baseline: 1.0893x; 1.0893x over previous
import functools

import jax
import jax.numpy as jnp
from jax import lax
from jax.experimental import pallas as pl
from jax.experimental.pallas import tpu as pltpu
from jax.experimental.pallas import tpu_sc as plsc

B = 16
MAX_LEN = 2048
D = 768
NW = 32
ROWS_PER_W = MAX_LEN // 2
CHUNK = 64
L = 16


def _pos_body(table_hbm, len_hbm, out_hbm, len_v, idx_v, buf, sem):
    wid = lax.axis_index("s") * 2 + lax.axis_index("c")
    b = wid // 2
    h = wid % 2

    pltpu.sync_copy(len_hbm.at[b], len_v)
    iota = lax.iota(jnp.int32, L)
    len_b = len_v[...]

    def chunk_body(g, carry):
        base = h * ROWS_PER_W + g * CHUNK
        for j in range(CHUNK // L):
            vec = base + j * L + iota + 1
            idx_v[pl.ds(j * L, L)] = jnp.where(vec <= len_b, vec, 0)
        pltpu.async_copy(table_hbm.at[idx_v], buf, sem).wait()
        pltpu.sync_copy(buf, out_hbm.at[b, pl.ds(base, CHUNK)])
        return carry

    lax.fori_loop(0, ROWS_PER_W // CHUNK, chunk_body, 0)


def kernel(input_len, table):
    len_i32 = jnp.broadcast_to(input_len.astype(jnp.int32)[:, None], (B, L))
    mesh = plsc.VectorSubcoreMesh(core_axis_name="c", subcore_axis_name="s")
    run = functools.partial(
        pl.kernel,
        mesh=mesh,
        out_type=jax.ShapeDtypeStruct((B, MAX_LEN, D), jnp.float32),
        scratch_types=[
            pltpu.VMEM((L,), jnp.int32),
            pltpu.VMEM((CHUNK,), jnp.int32),
            pltpu.VMEM((CHUNK, D), jnp.float32),
            pltpu.SemaphoreType.DMA,
        ],
    )(_pos_body)
    return run(table, len_i32)


# speedup vs baseline: 4.7111x; 4.3247x over previous
import functools

import jax
import jax.numpy as jnp
from jax import lax
from jax.experimental import pallas as pl
from jax.experimental.pallas import tpu as pltpu
from jax.experimental.pallas import tpu_sc as plsc

B = 16
MAX_LEN = 2048
D = 768
NW = 32
CHUNK = MAX_LEN // NW
L = 16


def _pos_body(table_hbm, len_hbm, out_hbm,
              len_v, idx_v, tbuf, zbuf, sem_g, sem_w):
    wid = lax.axis_index("s") * 2 + lax.axis_index("c")
    s = wid * CHUNK

    pltpu.sync_copy(len_hbm, len_v)
    lens = len_v[...]
    iota = lax.iota(jnp.int32, L)

    for j in range(CHUNK // L):
        idx_v[pl.ds(j * L, L)] = s + j * L + iota + 1
    pltpu.async_copy(table_hbm.at[idx_v], tbuf, sem_g).wait()
    for j in range(CHUNK // L):
        idx_v[pl.ds(j * L, L)] = jnp.zeros((L,), jnp.int32)
    pltpu.async_copy(table_hbm.at[idx_v], zbuf, sem_g).wait()

    n_async = jnp.int32(0)
    for b in range(B):
        lb = lens[b]

        @pl.when(s + CHUNK <= lb)
        def _():
            pltpu.async_copy(tbuf, out_hbm.at[b, pl.ds(s, CHUNK)], sem_w)

        @pl.when(lb <= s)
        def _():
            pltpu.async_copy(zbuf, out_hbm.at[b, pl.ds(s, CHUNK)], sem_w)

        outside = (s + CHUNK <= lb) | (lb <= s)
        n_async = n_async + jnp.where(outside, 1, 0).astype(jnp.int32)

    def drain(i, carry):
        @pl.when(i < n_async)
        def _():
            pltpu.make_async_copy(tbuf, out_hbm.at[0, pl.ds(0, CHUNK)],
                                  sem_w).wait()
        return carry

    lax.fori_loop(0, B, drain, 0)

    for b in range(B):
        lb = lens[b]

        @pl.when((s < lb) & (lb < s + CHUNK))
        def _():
            for j in range(CHUNK // L):
                vec = s + j * L + iota + 1
                idx_v[pl.ds(j * L, L)] = jnp.where(vec <= lb, vec, 0)
            pltpu.async_copy(table_hbm.at[idx_v], tbuf, sem_g).wait()
            pltpu.sync_copy(tbuf, out_hbm.at[b, pl.ds(s, CHUNK)])


def kernel(input_len, table):
    len_i32 = input_len.astype(jnp.int32)
    mesh = plsc.VectorSubcoreMesh(core_axis_name="c", subcore_axis_name="s")
    run = functools.partial(
        pl.kernel,
        mesh=mesh,
        out_type=jax.ShapeDtypeStruct((B, MAX_LEN, D), jnp.float32),
        scratch_types=[
            pltpu.VMEM((L,), jnp.int32),
            pltpu.VMEM((CHUNK,), jnp.int32),
            pltpu.VMEM((CHUNK, D), jnp.float32),
            pltpu.VMEM((CHUNK, D), jnp.float32),
            pltpu.SemaphoreType.DMA,
            pltpu.SemaphoreType.DMA,
        ],
    )(_pos_body)
    return run(table, len_i32)


# speedup vs baseline: 7.6594x; 1.6258x over previous
import functools

import jax
import jax.numpy as jnp
from jax import lax
from jax.experimental import pallas as pl
from jax.experimental.pallas import tpu as pltpu
from jax.experimental.pallas import tpu_sc as plsc

B = 16
MAX_LEN = 2048
D = 768
NW = 32
NS = 16
CHUNK = MAX_LEN // NW
L = 16


def _pos_body(table_hbm, len_hbm, out_hbm,
              len_v, idx_v, tbuf, sh_t, sh_z, sem_g, sem_w):
    cid = lax.axis_index("c")
    sid = lax.axis_index("s")
    wid = sid * 2 + cid
    s = wid * CHUNK

    pltpu.sync_copy(len_hbm, len_v)
    lens = len_v[...]
    iota = lax.iota(jnp.int32, L)

    @pl.when(sid == 0)
    def _():
        for j in range(CHUNK // L):
            idx_v[pl.ds(j * L, L)] = jnp.zeros((L,), jnp.int32)
        pltpu.async_copy(table_hbm.at[idx_v], tbuf, sem_g).wait()
        pltpu.sync_copy(tbuf, sh_z)

    for j in range(CHUNK // L):
        idx_v[pl.ds(j * L, L)] = s + j * L + iota + 1
    pltpu.async_copy(table_hbm.at[idx_v], tbuf, sem_g).wait()
    pltpu.sync_copy(tbuf, sh_t.at[sid])

    plsc.subcore_barrier()

    n_async = jnp.int32(0)
    for b in range(B):
        lb = lens[b]

        @pl.when(s + CHUNK <= lb)
        def _():
            pltpu.async_copy(sh_t.at[sid], out_hbm.at[b, pl.ds(s, CHUNK)],
                             sem_w)

        @pl.when(lb <= s)
        def _():
            pltpu.async_copy(sh_z, out_hbm.at[b, pl.ds(s, CHUNK)], sem_w)

        outside = (s + CHUNK <= lb) | (lb <= s)
        n_async = n_async + jnp.where(outside, 1, 0).astype(jnp.int32)

    def drain(i, carry):
        @pl.when(i < n_async)
        def _():
            pltpu.make_async_copy(sh_z, out_hbm.at[0, pl.ds(0, CHUNK)],
                                  sem_w).wait()
        return carry

    lax.fori_loop(0, B, drain, 0)

    for b in range(B):
        lb = lens[b]

        @pl.when((s < lb) & (lb < s + CHUNK))
        def _():
            for j in range(CHUNK // L):
                vec = s + j * L + iota + 1
                idx_v[pl.ds(j * L, L)] = jnp.where(vec <= lb, vec, 0)
            pltpu.async_copy(table_hbm.at[idx_v], tbuf, sem_g).wait()
            pltpu.sync_copy(tbuf, out_hbm.at[b, pl.ds(s, CHUNK)])


def kernel(input_len, table):
    len_i32 = input_len.astype(jnp.int32)
    mesh = plsc.VectorSubcoreMesh(core_axis_name="c", subcore_axis_name="s")
    run = functools.partial(
        pl.kernel,
        mesh=mesh,
        out_type=jax.ShapeDtypeStruct((B, MAX_LEN, D), jnp.float32),
        scratch_types=[
            pltpu.VMEM((L,), jnp.int32),
            pltpu.VMEM((CHUNK,), jnp.int32),
            pltpu.VMEM((CHUNK, D), jnp.float32),
            pltpu.VMEM_SHARED((NS, CHUNK, D), jnp.float32),
            pltpu.VMEM_SHARED((CHUNK, D), jnp.float32),
            pltpu.SemaphoreType.DMA,
            pltpu.SemaphoreType.DMA,
        ],
    )(_pos_body)
    return run(table, len_i32)
